# baseline (speedup 1.0000x reference)
import jax
import jax.numpy as jnp
from jax import lax
from jax.experimental import pallas as pl
from jax.experimental.pallas import tpu as pltpu, tpu_sc as plsc

NODE_NUM = 100
HIDDEN_SIZE = 4096
TOTAL = NODE_NUM * HIDDEN_SIZE

NUM_CORES = 2
NUM_SUBCORES = 16
NUM_WORKERS = NUM_CORES * NUM_SUBCORES
CHUNK = TOTAL // NUM_WORKERS


def _copy_body(in_hbm, out_hbm, buf):
    wid = lax.axis_index("s") * NUM_CORES + lax.axis_index("c")
    base = wid * CHUNK
    pltpu.sync_copy(in_hbm.at[pl.ds(base, CHUNK)], buf)
    pltpu.sync_copy(buf, out_hbm.at[pl.ds(base, CHUNK)])


def kernel(node_table):
    flat = node_table.reshape(TOTAL)
    mesh = plsc.VectorSubcoreMesh(core_axis_name="c", subcore_axis_name="s")
    out = pl.kernel(
        _copy_body,
        out_type=jax.ShapeDtypeStruct((TOTAL,), jnp.float32),
        scratch_types=[pltpu.VMEM((CHUNK,), jnp.float32)],
        mesh=mesh,
    )(flat)
    return out.reshape(NODE_NUM, HIDDEN_SIZE)
